# baseline (speedup 1.0000x reference)
import functools

import jax
import jax.numpy as jnp
from jax.experimental import pallas as pl
from jax.experimental.pallas import tpu as pltpu

_BETA = 0.25


def _rvq_block(x_ref, cb_ref, xq_ref, idx_ref, loss_ref, *, n_levels, k):
    r = x_ref[...]
    xq = jnp.zeros_like(r)
    loss_acc = jnp.float32(0.0)
    idx_cols = []
    for l in range(n_levels):
        cb = cb_ref[l]
        rn = jnp.sum(r * r, axis=1, keepdims=True)
        cn = jnp.sum(cb * cb, axis=1)[None, :]
        d = rn - 2.0 * jnp.dot(r, cb.T, preferred_element_type=jnp.float32) + cn
        dmin = jnp.min(d, axis=1, keepdims=True)
        iota = jax.lax.broadcasted_iota(jnp.int32, d.shape, 1)
        idx = jnp.min(jnp.where(d == dmin, iota, k), axis=1)
        onehot = (iota == idx[:, None]).astype(jnp.float32)
        zq = jnp.dot(onehot, cb, preferred_element_type=jnp.float32,
                     precision=jax.lax.Precision.HIGHEST)
        r = r - zq
        xq = xq + zq
        loss_acc = loss_acc + jnp.sum(r * r)
        idx_cols.append(idx)

    xq_ref[...] = xq
    idx_ref[...] = jnp.stack(idx_cols, axis=1)

    loss2d = loss_acc.reshape(1, 1)

    @pl.when(pl.program_id(0) == 0)
    def _init():
        loss_ref[...] = loss2d

    @pl.when(pl.program_id(0) != 0)
    def _acc():
        loss_ref[...] += loss2d


@jax.jit
def kernel(x, codebooks):
    n, e = x.shape
    n_levels, k, _ = codebooks.shape
    block = 512
    grid = (n // block,)

    xq, idx, loss = pl.pallas_call(
        functools.partial(_rvq_block, n_levels=n_levels, k=k),
        grid=grid,
        in_specs=[
            pl.BlockSpec((block, e), lambda i: (i, 0)),
            pl.BlockSpec((n_levels, k, e), lambda i: (0, 0, 0)),
        ],
        out_specs=[
            pl.BlockSpec((block, e), lambda i: (i, 0)),
            pl.BlockSpec((block, n_levels), lambda i: (i, 0)),
            pl.BlockSpec((1, 1), lambda i: (0, 0)),
        ],
        out_shape=[
            jax.ShapeDtypeStruct((n, e), jnp.float32),
            jax.ShapeDtypeStruct((n, n_levels), jnp.int32),
            jax.ShapeDtypeStruct((1, 1), jnp.float32),
        ],
    )(x, codebooks)

    mean_loss = loss[0, 0] * ((1.0 + _BETA) / (n_levels * n * e))
    return (xq, mean_loss, idx)


# speedup vs baseline: 1.5459x; 1.5459x over previous
import functools

import jax
import jax.numpy as jnp
from jax.experimental import pallas as pl
from jax.experimental.pallas import tpu as pltpu

_BETA = 0.25

_DIMNUMS = (((1,), (1,)), ((), ()))
_GATHER_DIMNUMS = (((1,), (0,)), ((), ()))


def _rvq_block(x_ref, cb_ref, cat_ref,
               xq_ref, idx_ref, loss_ref, *, n_levels, k):
    r = x_ref[...]
    xq = jnp.zeros_like(r)
    loss_acc = jnp.float32(0.0)
    idx_cols = []
    for l in range(n_levels):
        cb = cb_ref[l]
        rn = jnp.sum(r * r, axis=1, keepdims=True)
        cn = jnp.sum(cb * cb, axis=1)[None, :]
        prod = jnp.dot(r, cb.T, preferred_element_type=jnp.float32)
        d = rn - 2.0 * prod + cn
        dmin = jnp.min(d, axis=1, keepdims=True)
        iota = jax.lax.broadcasted_iota(jnp.int32, d.shape, 1)
        idx = jnp.min(jnp.where(d == dmin, iota, k), axis=1)
        onehot = (iota == idx[:, None]).astype(jnp.float32)
        g = jax.lax.dot_general(onehot, cat_ref[l], _GATHER_DIMNUMS,
                                preferred_element_type=jnp.float32)
        e = cb.shape[1]
        zq = (g[:, :e] + g[:, e:2 * e]) + g[:, 2 * e:]
        r = r - zq
        xq = xq + zq
        loss_acc = loss_acc + jnp.sum(r * r)
        idx_cols.append(idx)

    xq_ref[...] = xq
    idx_ref[...] = jnp.stack(idx_cols, axis=1)
    loss2d = loss_acc.reshape(1, 1)

    @pl.when(pl.program_id(0) == 0)
    def _init():
        loss_ref[...] = loss2d

    @pl.when(pl.program_id(0) != 0)
    def _acc():
        loss_ref[...] += loss2d


@jax.jit
def kernel(x, codebooks):
    n, e = x.shape
    n_levels, k, _ = codebooks.shape
    block = 512
    grid = (n // block,)

    bits = jax.lax.bitcast_convert_type(codebooks, jnp.int32)
    hi = jax.lax.bitcast_convert_type(bits & jnp.int32(-65536), jnp.float32)
    rem = codebooks - hi
    bits2 = jax.lax.bitcast_convert_type(rem, jnp.int32)
    mid = jax.lax.bitcast_convert_type(bits2 & jnp.int32(-65536), jnp.float32)
    lo = rem - mid
    cat = jnp.concatenate([hi, mid, lo], axis=2)

    xq, idx, loss = pl.pallas_call(
        functools.partial(_rvq_block, n_levels=n_levels, k=k),
        grid=grid,
        in_specs=[
            pl.BlockSpec((block, e), lambda i: (i, 0)),
            pl.BlockSpec((n_levels, k, e), lambda i: (0, 0, 0)),
            pl.BlockSpec((n_levels, k, 3 * e), lambda i: (0, 0, 0)),
        ],
        out_specs=[
            pl.BlockSpec((block, e), lambda i: (i, 0)),
            pl.BlockSpec((block, n_levels), lambda i: (i, 0)),
            pl.BlockSpec((1, 1), lambda i: (0, 0)),
        ],
        out_shape=[
            jax.ShapeDtypeStruct((n, e), jnp.float32),
            jax.ShapeDtypeStruct((n, n_levels), jnp.int32),
            jax.ShapeDtypeStruct((1, 1), jnp.float32),
        ],
    )(x, codebooks, cat)

    mean_loss = loss[0, 0] * ((1.0 + _BETA) / (n_levels * n * e))
    return (xq, mean_loss, idx)
